# baseline (speedup 1.0000x reference)
import functools

import jax
import jax.numpy as jnp
from jax import lax
from jax.experimental import pallas as pl
from jax.experimental.pallas import tpu as pltpu
from jax.experimental.pallas import tpu_sc as plsc

_BATCH = 4096

_INFO = plsc.get_sparse_core_info()
_NW = _INFO.num_cores * _INFO.num_subcores
_BPW = _BATCH // _NW

_MESH = plsc.VectorSubcoreMesh(core_axis_name="c", subcore_axis_name="s")


@functools.partial(
    pl.kernel,
    mesh=_MESH,
    out_type=(
        jax.ShapeDtypeStruct((_BATCH, 6, 6), jnp.float32),
        jax.ShapeDtypeStruct((_BATCH, 6, 3), jnp.float32),
    ),
    scratch_types=[
        pltpu.VMEM((_BPW + 16,), jnp.int32),
        pltpu.SemaphoreType.DMA,
        pltpu.SemaphoreType.DMA,
    ],
)
def _gather_rows(idx_hbm, rot_hbm, tra_hbm, rot_out, tra_out,
                 idx_v, sem_r, sem_t):
    wid = lax.axis_index("s") * _INFO.num_cores + lax.axis_index("c")
    base = wid * _BPW
    pltpu.sync_copy(idx_hbm.at[pl.ds(base, _BPW)], idx_v.at[pl.ds(0, _BPW)])

    def fire(i, _):
        idx = idx_v[pl.ds(i, 16)][0]
        pltpu.async_copy(rot_hbm.at[idx], rot_out.at[base + i], sem_r)
        pltpu.async_copy(tra_hbm.at[idx], tra_out.at[base + i], sem_t)
        return ()

    lax.fori_loop(0, _BPW, fire, ())

    def drain(i, _):
        pltpu.make_async_copy(rot_hbm.at[0], rot_out.at[base + i],
                              sem_r).wait()
        pltpu.make_async_copy(tra_hbm.at[0], tra_out.at[base + i],
                              sem_t).wait()
        return ()

    lax.fori_loop(0, _BPW, drain, ())


def kernel(indexes, rotation_table, translation_table):
    return _gather_rows(indexes, rotation_table, translation_table)


# speedup vs baseline: 2.9424x; 2.9424x over previous
import functools

import jax
import jax.numpy as jnp
from jax import lax
from jax.experimental import pallas as pl
from jax.experimental.pallas import tpu as pltpu

_BATCH = 4096
_R = 512
_NCH = _BATCH // _R
_W = 128


def _gather_body(idx_s, rot_any, tra_any, rot_o, tra_o,
                 rot_v, tra_v, sem_r, sem_t, sem_w):
    def chunk(c, _):
        base = c * _R

        def fire(i, _):
            idx = idx_s[base + i]
            pltpu.make_async_copy(rot_any.at[idx], rot_v.at[i], sem_r).start()
            pltpu.make_async_copy(tra_any.at[idx], tra_v.at[i], sem_t).start()

            @pl.when(i >= _W)
            def _():
                j = i - _W
                pltpu.make_async_copy(rot_any.at[0], rot_v.at[j],
                                      sem_r).wait()
                pltpu.make_async_copy(tra_any.at[0], tra_v.at[j],
                                      sem_t).wait()
            return ()

        lax.fori_loop(0, _R, fire, ())

        def drain(i, _):
            pltpu.make_async_copy(rot_any.at[0], rot_v.at[i], sem_r).wait()
            pltpu.make_async_copy(tra_any.at[0], tra_v.at[i], sem_t).wait()
            return ()

        lax.fori_loop(_R - _W, _R, drain, ())

        pltpu.make_async_copy(rot_v, rot_o.at[pl.ds(base, _R)], sem_w).start()
        pltpu.make_async_copy(rot_v, rot_o.at[pl.ds(base, _R)], sem_w).wait()
        pltpu.make_async_copy(tra_v, tra_o.at[pl.ds(base, _R)], sem_w).start()
        pltpu.make_async_copy(tra_v, tra_o.at[pl.ds(base, _R)], sem_w).wait()
        return ()

    lax.fori_loop(0, _NCH, chunk, ())


@jax.jit
def kernel(indexes, rotation_table, translation_table):
    grid_spec = pltpu.PrefetchScalarGridSpec(
        num_scalar_prefetch=1,
        grid=(1,),
        in_specs=[
            pl.BlockSpec(memory_space=pl.ANY),
            pl.BlockSpec(memory_space=pl.ANY),
        ],
        out_specs=[
            pl.BlockSpec(memory_space=pl.ANY),
            pl.BlockSpec(memory_space=pl.ANY),
        ],
        scratch_shapes=[
            pltpu.VMEM((_R, 6, 6), jnp.float32),
            pltpu.VMEM((_R, 6, 3), jnp.float32),
            pltpu.SemaphoreType.DMA,
            pltpu.SemaphoreType.DMA,
            pltpu.SemaphoreType.DMA,
        ],
    )
    rot, tra = pl.pallas_call(
        _gather_body,
        grid_spec=grid_spec,
        out_shape=[
            jax.ShapeDtypeStruct((_BATCH, 6, 6), jnp.float32),
            jax.ShapeDtypeStruct((_BATCH, 6, 3), jnp.float32),
        ],
    )(indexes, rotation_table, translation_table)
    return (rot, tra)


# speedup vs baseline: 3.2152x; 1.0927x over previous
import functools

import jax
import jax.numpy as jnp
from jax import lax
from jax.experimental import pallas as pl
from jax.experimental.pallas import tpu as pltpu
from jax.experimental.pallas import tpu_sc as plsc

_BATCH = 4096

_INFO = plsc.get_sparse_core_info()
_NW = _INFO.num_cores * _INFO.num_subcores
_BPW = _BATCH // _NW
_CH = 32
_NCH = _BPW // _CH
_NS = 4
_SUB = _CH // _NS

_MESH = plsc.VectorSubcoreMesh(core_axis_name="c", subcore_axis_name="s")


@functools.partial(
    pl.kernel,
    mesh=_MESH,
    out_type=(
        jax.ShapeDtypeStruct((_BATCH, 6, 6), jnp.float32),
        jax.ShapeDtypeStruct((_BATCH, 6, 3), jnp.float32),
    ),
    scratch_types=[
        pltpu.VMEM((_BPW + 16,), jnp.int32),
        pltpu.VMEM((_CH, 6, 6), jnp.float32),
        pltpu.VMEM((_CH, 6, 3), jnp.float32),
        [pltpu.SemaphoreType.DMA] * _NS,
        [pltpu.SemaphoreType.DMA] * _NS,
        [pltpu.SemaphoreType.DMA] * 2,
    ],
)
def _gather_rows(idx_hbm, rot_hbm, tra_hbm, rot_out, tra_out,
                 idx_v, rot_v, tra_v, sems_r, sems_t, sems_w):
    wid = lax.axis_index("s") * _INFO.num_cores + lax.axis_index("c")
    base = wid * _BPW
    pltpu.sync_copy(idx_hbm.at[pl.ds(base, _BPW)], idx_v.at[pl.ds(0, _BPW)])

    def chunk(c, _):
        cb = c * _CH

        for k in range(_NS):
            def fire(j, _):
                i = j * _NS + k
                idx = idx_v[pl.ds(cb + i, 16)][0]
                pltpu.async_copy(rot_hbm.at[idx], rot_v.at[i], sems_r[k])
                pltpu.async_copy(tra_hbm.at[idx], tra_v.at[i], sems_t[k])
                return ()

            lax.fori_loop(0, _SUB, fire, ())

        for k in range(_NS):
            def drain(j, _):
                i = j * _NS + k
                pltpu.make_async_copy(rot_hbm.at[0], rot_v.at[i],
                                      sems_r[k]).wait()
                pltpu.make_async_copy(tra_hbm.at[0], tra_v.at[i],
                                      sems_t[k]).wait()
                return ()

            lax.fori_loop(0, _SUB, drain, ())

        cr = pltpu.async_copy(rot_v, rot_out.at[pl.ds(base + cb, _CH)],
                              sems_w[0])
        ct = pltpu.async_copy(tra_v, tra_out.at[pl.ds(base + cb, _CH)],
                              sems_w[1])
        cr.wait()
        ct.wait()
        return ()

    lax.fori_loop(0, _NCH, chunk, ())


def kernel(indexes, rotation_table, translation_table):
    return _gather_rows(indexes, rotation_table, translation_table)
